# Initial kernel scaffold; baseline (speedup 1.0000x reference)
import jax
import jax.numpy as jnp
from jax.experimental import pallas as pl


def kernel(x_node, x_edge, edge_index, W1e, b1e, W2e, b2e, W1n, b1n, W2n, b2n):
    raise NotImplementedError("write your pallas kernel here")


# speedup vs baseline: 12.8400x; 12.8400x over previous
import functools

import jax
import jax.numpy as jnp
from jax import lax
from jax.experimental import pallas as pl
from jax.experimental.pallas import tpu as pltpu
from jax.experimental.pallas import tpu_sc as plsc

N = 100000
E = 1600000
D_NODE = 32
D_EDGE = 6

W = 128
HALF = N // 2
ACC_ROWS = 50176
TRASH = HALF


def _sc_gather(x_node, idx_flat):
    m = idx_flat.shape[1]
    mesh = plsc.VectorSubcoreMesh(core_axis_name="core", subcore_axis_name="subcore")

    @functools.partial(
        pl.kernel,
        out_type=jax.ShapeDtypeStruct((m, D_NODE), jnp.float32),
        mesh=mesh,
        compiler_params=pltpu.CompilerParams(use_tc_tiling_on_sc=False),
    )
    def k(x_hbm, i_hbm, o_hbm):
        def body(i_vmem, o_vmem):
            pltpu.sync_copy(x_hbm.at[i_vmem.at[0]], o_vmem)

        pltpu.emit_pipeline(
            body,
            grid=(m // W,),
            in_specs=[pl.BlockSpec((1, W), lambda i: (0, i))],
            out_specs=[pl.BlockSpec((W, D_NODE), lambda i: (i, 0))],
            core_axis_name=("core", "subcore"),
            dimension_semantics=(pltpu.PARALLEL,),
        )(i_hbm, o_hbm)

    return k(x_node, idx_flat)


def _tc_mlp(gp, xe_p, w1d, w1s, w1e, b1e, w2e, b2e, wnd, wnm, b1n, w2n, b2n):
    E4 = E // 4
    Bq = 2000
    nb = E4 // Bq

    def body(xs_ref, xd_ref, xe_ref, w1d_ref, w1s_ref, w1e_ref, b1e_ref,
             w2e_ref, b2e_ref, wnd_ref, wnm_ref, b1n_ref, w2n_ref, b2n_ref,
             em_ref, nm_ref):
        xs = xs_ref[...]
        xd = xd_ref[...]
        xe = xe_ref[...]
        dot = functools.partial(jnp.dot, preferred_element_type=jnp.float32)
        h = jnp.maximum(
            dot(xd, w1d_ref[...]) + dot(xs, w1s_ref[...]) + dot(xe, w1e_ref[...])
            + b1e_ref[...], 0.0)
        em = jnp.maximum(dot(h, w2e_ref[...]) + b2e_ref[...], 0.0)
        h2 = jnp.maximum(
            dot(xd, wnd_ref[...]) + dot(em, wnm_ref[...]) + b1n_ref[...], 0.0)
        nm = jnp.maximum(dot(h2, w2n_ref[...]) + b2n_ref[...], 0.0)
        em_ref[...] = em
        nm_ref[...] = nm

    full = lambda a: pl.BlockSpec(a.shape, lambda i: (0,) * a.ndim)
    return pl.pallas_call(
        body,
        grid=(nb,),
        in_specs=[
            pl.BlockSpec((Bq, 128), lambda i: (i, 0)),
            pl.BlockSpec((Bq, 128), lambda i: (nb + i, 0)),
            pl.BlockSpec((Bq, 4 * D_EDGE), lambda i: (i, 0)),
            full(w1d), full(w1s), full(w1e), full(b1e),
            full(w2e), full(b2e), full(wnd), full(wnm),
            full(b1n), full(w2n), full(b2n),
        ],
        out_specs=[
            pl.BlockSpec((Bq, 4 * D_EDGE), lambda i: (i, 0)),
            pl.BlockSpec((Bq, 128), lambda i: (i, 0)),
        ],
        out_shape=[
            jax.ShapeDtypeStruct((E4, 4 * D_EDGE), jnp.float32),
            jax.ShapeDtypeStruct((E4, 128), jnp.float32),
        ],
    )(gp, gp, xe_p, w1d, w1s, w1e, b1e, w2e, b2e, wnd, wnm, b1n, w2n, b2n)


def _sc_scatter(node_msg, dst2d, zeros):
    mesh = plsc.VectorSubcoreMesh(core_axis_name="core", subcore_axis_name="subcore")
    rows_per_sub_acc = ACC_ROWS // 16
    rows_per_sub_out = HALF // 16
    @functools.partial(
        pl.kernel,
        out_type=jax.ShapeDtypeStruct((N, D_NODE), jnp.float32),
        mesh=mesh,
        scratch_types=[
            pltpu.VMEM_SHARED((ACC_ROWS, D_NODE), jnp.float32),
            pltpu.VMEM((1, W), jnp.int32),
        ],
        compiler_params=pltpu.CompilerParams(use_tc_tiling_on_sc=False),
    )
    def k(msg_hbm, dst_hbm, z_hbm, nm_hbm, acc, idx_loc):
        core = lax.axis_index("core")
        sub = lax.axis_index("subcore")
        pltpu.sync_copy(
            z_hbm.at[pl.ds(sub * rows_per_sub_acc, rows_per_sub_acc)],
            acc.at[pl.ds(sub * rows_per_sub_acc, rows_per_sub_acc)],
        )
        plsc.subcore_barrier()

        base = core * HALF

        def body(msg_vmem, dst_vmem):
            for kk in range(W // 16):
                v = dst_vmem[0, pl.ds(kk * 16, 16)]
                lo = v - base
                ok = (lo >= 0) & (lo < HALF)
                idx_loc[0, pl.ds(kk * 16, 16)] = jnp.where(ok, lo, TRASH)
            pltpu.sync_copy(msg_vmem, acc.at[idx_loc.at[0]], add=True)

        pltpu.emit_pipeline(
            body,
            grid=(E // W,),
            in_specs=[
                pl.BlockSpec((W, D_NODE), lambda i: (i, 0)),
                pl.BlockSpec((1, W), lambda i: (0, i)),
            ],
            out_specs=[],
            core_axis_name="subcore",
            dimension_semantics=(pltpu.PARALLEL,),
        )(msg_hbm, dst_hbm)
        plsc.subcore_barrier()

        pltpu.sync_copy(
            acc.at[pl.ds(sub * rows_per_sub_out, rows_per_sub_out)],
            nm_hbm.at[pl.ds(base + sub * rows_per_sub_out, rows_per_sub_out)],
        )

    return k(node_msg, dst2d, zeros)


def _blkdiag4(w):
    return jax.scipy.linalg.block_diag(w, w, w, w)


def kernel(x_node, x_edge, edge_index, W1e, b1e, W2e, b2e, W1n, b1n, W2n, b2n):
    E4 = E // 4
    edge_index = edge_index.astype(jnp.int32)
    idx_flat = edge_index.reshape(1, 2 * E)
    g = _sc_gather(x_node, idx_flat)
    gp = g.reshape(2 * E4, 128)

    xe_p = x_edge.reshape(E4, 4 * D_EDGE)

    tile4 = lambda b: jnp.tile(b.reshape(1, -1), (1, 4))
    em_p, nm_p = _tc_mlp(
        gp, xe_p,
        _blkdiag4(W1e[:D_NODE]), _blkdiag4(W1e[D_NODE:2 * D_NODE]),
        _blkdiag4(W1e[2 * D_NODE:]), tile4(b1e),
        _blkdiag4(W2e), tile4(b2e),
        _blkdiag4(W1n[:D_NODE]), _blkdiag4(W1n[D_NODE:]), tile4(b1n),
        _blkdiag4(W2n), tile4(b2n),
    )

    em = em_p.reshape(E, D_EDGE)
    node_msg = nm_p.reshape(E, D_NODE)
    dst2d = edge_index[1].reshape(1, E)
    zeros = jnp.zeros((ACC_ROWS, D_NODE), jnp.float32)
    nm = _sc_scatter(node_msg, dst2d, zeros)
    return (nm, em)
